# baseline (speedup 1.0000x reference)
import math

import jax
import jax.numpy as jnp
from jax.experimental import pallas as pl

_TS = 20
_JPB = 4
_NHID = 32
_ATT_DOT = 32
_NFEAT = 16
_ALPHA = 0.2
_TARGET = 7
_SCALE = 1.0 / math.sqrt(_ATT_DOT)


def _one_step(c, s, ext_ref, k, wh, we_t, wq, wk):
    wh_t = wh.T

    u = jax.lax.dot_general(wh, wq, (((1,), (0,)), ((), ())),
                            preferred_element_type=jnp.float32)
    v = jax.lax.dot_general(wh, wk, (((1,), (0,)), ((), ())),
                            preferred_element_type=jnp.float32)
    a_c0 = jnp.sum(u * v[0:1, :], axis=1, keepdims=True)
    a_c1 = jnp.sum(u * v[1:2, :], axis=1, keepdims=True)

    c7 = c[_TARGET:_TARGET + 1, :]
    s7 = s[_TARGET:_TARGET + 1, :]
    raw = (c7 * (a_c0[0:1, :] * c + a_c1[0:1, :] * s)
           + s7 * (a_c0[1:2, :] * c + a_c1[1:2, :] * s))
    logits = jnp.where(raw >= 0, raw, raw * _ALPHA) * _SCALE

    mx = jnp.max(logits, axis=0, keepdims=True)
    p = jnp.exp(logits - mx)
    attn = p * (1.0 / jnp.sum(p, axis=0, keepdims=True))

    wc = jnp.sum(attn * c, axis=0, keepdims=True)
    ws = jnp.sum(attn * s, axis=0, keepdims=True)
    on_t = wh_t[:, 0:1] * wc + wh_t[:, 1:2] * ws

    en_t = jnp.zeros_like(on_t)
    for f in range(_NFEAT):
        ef = ext_ref[k, f].astype(jnp.float32)
        g_f = jnp.sum(attn * ef, axis=0, keepdims=True)
        en_t = en_t + we_t[:, f:f + 1] * g_f
    return on_t, en_t


def _gat_step(xr_ref, xc_ref, side_ref, ext_ref, wh_ref, we_ref, wq_ref,
              wk_ref, out_ref):
    for k in range(_JPB):
        c = jnp.concatenate([xr_ref[:, k, 0, :], xc_ref[k]],
                            axis=0).astype(jnp.float32)
        s = side_ref[:, k, 0, :].astype(jnp.float32)
        on_t, en_t = _one_step(c, s, ext_ref, k, wh_ref[k], we_ref[k].T,
                               wq_ref[k], wk_ref[k])
        out_ref[0, k] = on_t
        out_ref[1, k] = en_t


def kernel(x_crime, x_regions, x_ext, s_crime, W_h, W_e, Wq, Wk):
    B, T = x_crime.shape
    N = x_ext.shape[0]
    t0 = T - _TS
    nsteps = _TS // _JPB

    xc_t = x_crime.T.reshape(T, 1, B)
    xr4 = x_regions.reshape(N - 1, T, 1, B)
    sc4 = s_crime.reshape(N, T, 1, B)
    ext_nf = x_ext[:, t0:, :, :].astype(jnp.bfloat16).transpose(1, 0, 3, 2)
    ext_nf = jax.lax.optimization_barrier(ext_nf)
    ext_sl = ext_nf.transpose(0, 2, 1, 3)

    out = pl.pallas_call(
        _gat_step,
        grid=(nsteps,),
        in_specs=[
            pl.BlockSpec((N - 1, _JPB, 1, B),
                         lambda j: (0, t0 // _JPB + j, 0, 0)),
            pl.BlockSpec((_JPB, 1, B), lambda j: (t0 // _JPB + j, 0, 0)),
            pl.BlockSpec((N, _JPB, 1, B),
                         lambda j: (0, t0 // _JPB + j, 0, 0)),
            pl.BlockSpec((_JPB, _NFEAT, N, B), lambda j: (j, 0, 0, 0)),
            pl.BlockSpec((_JPB, 2, _NHID), lambda j: (j, 0, 0)),
            pl.BlockSpec((_JPB, _NFEAT, _NHID), lambda j: (j, 0, 0)),
            pl.BlockSpec((_JPB, _NHID, _ATT_DOT), lambda j: (j, 0, 0)),
            pl.BlockSpec((_JPB, _NHID, _ATT_DOT), lambda j: (j, 0, 0)),
        ],
        out_specs=pl.BlockSpec((2, _JPB, _NHID, B), lambda j: (0, j, 0, 0)),
        out_shape=jax.ShapeDtypeStruct((2, _TS, _NHID, B), jnp.float32),
    )(xr4, xc_t, sc4, ext_sl, W_h, W_e, Wq, Wk)

    return out.transpose(3, 1, 2, 0)


# speedup vs baseline: 1.0559x; 1.0559x over previous
import math

import jax
import jax.numpy as jnp
from jax.experimental import pallas as pl

_TS = 20
_JPB = 4
_NHID = 32
_ATT_DOT = 32
_NFEAT = 16
_ALPHA = 0.2
_TARGET = 7
_SCALE = 1.0 / math.sqrt(_ATT_DOT)


def _one_step(c, s, ext_ref, k, wh, we_t, wq, wk):
    wh_t = wh.T

    u = jax.lax.dot_general(wh, wq, (((1,), (0,)), ((), ())),
                            preferred_element_type=jnp.float32)
    v = jax.lax.dot_general(wh, wk, (((1,), (0,)), ((), ())),
                            preferred_element_type=jnp.float32)
    a_c0 = jnp.sum(u * v[0:1, :], axis=1, keepdims=True)
    a_c1 = jnp.sum(u * v[1:2, :], axis=1, keepdims=True)

    c7 = c[_TARGET:_TARGET + 1, :]
    s7 = s[_TARGET:_TARGET + 1, :]
    raw = (c7 * (a_c0[0:1, :] * c + a_c1[0:1, :] * s)
           + s7 * (a_c0[1:2, :] * c + a_c1[1:2, :] * s))
    logits = jnp.where(raw >= 0, raw, raw * _ALPHA) * _SCALE

    mx = jnp.max(logits, axis=0, keepdims=True)
    p = jnp.exp(logits - mx)
    attn = p * (1.0 / jnp.sum(p, axis=0, keepdims=True))

    wc = jnp.sum(attn * c, axis=0, keepdims=True)
    ws = jnp.sum(attn * s, axis=0, keepdims=True)
    on_t = wh_t[:, 0:1] * wc + wh_t[:, 1:2] * ws

    en_t = jnp.zeros_like(on_t)
    for f in range(_NFEAT):
        ef = ext_ref[k, f].astype(jnp.float32)
        g_f = jnp.sum(attn * ef, axis=0, keepdims=True)
        en_t = en_t + we_t[:, f:f + 1] * g_f
    return on_t, en_t


def _gat_step(xr_ref, xc_ref, side_ref, ext_ref, wh_ref, we_ref, wq_ref,
              wk_ref, out_ref):
    for k in range(_JPB):
        c = jnp.concatenate([xr_ref[:, k, 0, :], xc_ref[k]],
                            axis=0).astype(jnp.float32)
        s = side_ref[:, k, 0, :].astype(jnp.float32)
        on_t, en_t = _one_step(c, s, ext_ref, k, wh_ref[k], we_ref[k].T,
                               wq_ref[k], wk_ref[k])
        out_ref[0, k] = on_t
        out_ref[1, k] = en_t


def kernel(x_crime, x_regions, x_ext, s_crime, W_h, W_e, Wq, Wk):
    B, T = x_crime.shape
    N = x_ext.shape[0]
    t0 = T - _TS
    nsteps = _TS // _JPB

    xc_t = x_crime.T.reshape(T, 1, B)
    xr4 = x_regions.reshape(N - 1, T, 1, B)
    sc4 = s_crime.reshape(N, T, 1, B)
    ext_sl = (x_ext[:, t0:, :, :].astype(jnp.int8)
              .transpose(1, 3, 0, 2))

    out = pl.pallas_call(
        _gat_step,
        grid=(nsteps,),
        in_specs=[
            pl.BlockSpec((N - 1, _JPB, 1, B),
                         lambda j: (0, t0 // _JPB + j, 0, 0)),
            pl.BlockSpec((_JPB, 1, B), lambda j: (t0 // _JPB + j, 0, 0)),
            pl.BlockSpec((N, _JPB, 1, B),
                         lambda j: (0, t0 // _JPB + j, 0, 0)),
            pl.BlockSpec((_JPB, _NFEAT, N, B), lambda j: (j, 0, 0, 0)),
            pl.BlockSpec((_JPB, 2, _NHID), lambda j: (j, 0, 0)),
            pl.BlockSpec((_JPB, _NFEAT, _NHID), lambda j: (j, 0, 0)),
            pl.BlockSpec((_JPB, _NHID, _ATT_DOT), lambda j: (j, 0, 0)),
            pl.BlockSpec((_JPB, _NHID, _ATT_DOT), lambda j: (j, 0, 0)),
        ],
        out_specs=pl.BlockSpec((2, _JPB, _NHID, B), lambda j: (0, j, 0, 0)),
        out_shape=jax.ShapeDtypeStruct((2, _TS, _NHID, B), jnp.float32),
    )(xr4, xc_t, sc4, ext_sl, W_h, W_e, Wq, Wk)

    return out.transpose(3, 1, 2, 0)


# speedup vs baseline: 1.0688x; 1.0122x over previous
import math

import jax
import jax.numpy as jnp
from jax.experimental import pallas as pl

_TS = 20
_JPB = 4
_NHID = 32
_ATT_DOT = 32
_NFEAT = 16
_ALPHA = 0.2
_TARGET = 7
_SCALE = 1.0 / math.sqrt(_ATT_DOT)


def _one_step(c, s, ext_ref, k, wh, we_t, wq, wk):
    wh_t = wh.T

    u = jax.lax.dot_general(wh, wq, (((1,), (0,)), ((), ())),
                            preferred_element_type=jnp.float32)
    v = jax.lax.dot_general(wh, wk, (((1,), (0,)), ((), ())),
                            preferred_element_type=jnp.float32)
    a_c0 = jnp.sum(u * v[0:1, :], axis=1, keepdims=True)
    a_c1 = jnp.sum(u * v[1:2, :], axis=1, keepdims=True)

    c7 = c[_TARGET:_TARGET + 1, :]
    s7 = s[_TARGET:_TARGET + 1, :]
    raw = (c7 * (a_c0[0:1, :] * c + a_c1[0:1, :] * s)
           + s7 * (a_c0[1:2, :] * c + a_c1[1:2, :] * s))
    logits = jnp.where(raw >= 0, raw, raw * _ALPHA) * _SCALE

    mx = jnp.max(logits, axis=0, keepdims=True)
    p = jnp.exp(logits - mx)
    attn = p * (1.0 / jnp.sum(p, axis=0, keepdims=True))

    wc = jnp.sum(attn * c, axis=0, keepdims=True)
    ws = jnp.sum(attn * s, axis=0, keepdims=True)
    on_t = wh_t[:, 0:1] * wc + wh_t[:, 1:2] * ws

    efk = ext_ref[k].astype(jnp.float32)
    en_t = jnp.zeros_like(on_t)
    for f in range(_NFEAT):
        g_f = jnp.sum(attn * efk[:, f, :], axis=0, keepdims=True)
        en_t = en_t + we_t[:, f:f + 1] * g_f
    return on_t, en_t


def _gat_step(xr_ref, xc_ref, side_ref, ext_ref, wh_ref, we_ref, wq_ref,
              wk_ref, out_ref):
    for k in range(_JPB):
        c = jnp.concatenate([xr_ref[:, k, 0, :], xc_ref[k]],
                            axis=0).astype(jnp.float32)
        s = side_ref[:, k, 0, :].astype(jnp.float32)
        on_t, en_t = _one_step(c, s, ext_ref, k, wh_ref[k], we_ref[k].T,
                               wq_ref[k], wk_ref[k])
        out_ref[0, k] = on_t
        out_ref[1, k] = en_t


def kernel(x_crime, x_regions, x_ext, s_crime, W_h, W_e, Wq, Wk):
    B, T = x_crime.shape
    N = x_ext.shape[0]
    t0 = T - _TS
    nsteps = _TS // _JPB

    xc_t = x_crime.T.reshape(T, 1, B)
    xr4 = x_regions.reshape(N - 1, T, 1, B)
    sc4 = s_crime.reshape(N, T, 1, B)
    ext_sl = (x_ext[:, t0:, :, :].astype(jnp.int8)
              .transpose(1, 0, 3, 2))

    out = pl.pallas_call(
        _gat_step,
        grid=(nsteps,),
        in_specs=[
            pl.BlockSpec((N - 1, _JPB, 1, B),
                         lambda j: (0, t0 // _JPB + j, 0, 0)),
            pl.BlockSpec((_JPB, 1, B), lambda j: (t0 // _JPB + j, 0, 0)),
            pl.BlockSpec((N, _JPB, 1, B),
                         lambda j: (0, t0 // _JPB + j, 0, 0)),
            pl.BlockSpec((_JPB, N, _NFEAT, B), lambda j: (j, 0, 0, 0)),
            pl.BlockSpec((_JPB, 2, _NHID), lambda j: (j, 0, 0)),
            pl.BlockSpec((_JPB, _NFEAT, _NHID), lambda j: (j, 0, 0)),
            pl.BlockSpec((_JPB, _NHID, _ATT_DOT), lambda j: (j, 0, 0)),
            pl.BlockSpec((_JPB, _NHID, _ATT_DOT), lambda j: (j, 0, 0)),
        ],
        out_specs=pl.BlockSpec((2, _JPB, _NHID, B), lambda j: (0, j, 0, 0)),
        out_shape=jax.ShapeDtypeStruct((2, _TS, _NHID, B), jnp.float32),
    )(xr4, xc_t, sc4, ext_sl, W_h, W_e, Wq, Wk)

    return out.transpose(3, 1, 2, 0)
